# baseline (speedup 1.0000x reference)
import functools

import jax
import jax.numpy as jnp
from jax import lax
from jax.experimental import pallas as pl
from jax.experimental.pallas import tpu as pltpu
from jax.experimental.pallas import tpu_sc as plsc

ADIM = 32
BDIM = 16
NC, NS = 2, 16
NW = NC * NS
CH = 128

_f32 = jnp.float32
_SC_PARAMS = pltpu.CompilerParams(use_tc_tiling_on_sc=False)
_SC_PARAMS_TILED = pltpu.CompilerParams(use_tc_tiling_on_sc=True)


def _make_gather(bn: int, be: int):
    per_w = be // NW
    nch = per_w // CH
    mesh = plsc.VectorSubcoreMesh(core_axis_name="c", subcore_axis_name="s")

    @functools.partial(
        pl.kernel,
        out_type=jax.ShapeDtypeStruct((be, 128), jnp.bfloat16),
        mesh=mesh,
        scratch_types=[
            pltpu.VMEM((nch, CH), jnp.int32),
            pltpu.VMEM((per_w, ADIM), jnp.bfloat16),
            pltpu.SemaphoreType.DMA,
        ],
        compiler_params=_SC_PARAMS,
    )
    def gather_kernel(table_hbm, idx_hbm, out_hbm, idx_v, rows_v, sem):
        wid = lax.axis_index("s") * NC + lax.axis_index("c")
        pltpu.sync_copy(idx_hbm.at[wid], idx_v)
        copies = []
        for j in range(nch):
            copies.append(
                pltpu.async_copy(
                    table_hbm.at[idx_v.at[j]],
                    rows_v.at[pl.ds(j * CH, CH)],
                    sem,
                )
            )
        for c in copies:
            c.wait()
        pltpu.sync_copy(
            rows_v, out_hbm.at[pl.ds(wid * per_w, per_w), pl.ds(0, ADIM)]
        )

    return gather_kernel


def _make_scatter(bn: int, be: int):
    per_w = be // NW
    nch = per_w // CH
    rows_per_s = bn // NS
    zrows = 64
    mesh = plsc.VectorSubcoreMesh(core_axis_name="c", subcore_axis_name="s")

    @functools.partial(
        pl.kernel,
        out_type=jax.ShapeDtypeStruct((2 * bn, 128), _f32),
        mesh=mesh,
        scratch_types=[
            pltpu.VMEM((nch, CH), jnp.int32),
            pltpu.VMEM((per_w, ADIM), _f32),
            pltpu.VMEM((zrows, ADIM), _f32),
            pltpu.VMEM_SHARED((bn, ADIM), _f32),
            pltpu.SemaphoreType.DMA,
            pltpu.SemaphoreType.DMA,
        ],
        compiler_params=_SC_PARAMS,
    )
    def scatter_kernel(msg_hbm, idx_hbm, out_hbm, idx_v, rows_v, zbuf, acc, sem, sem2):
        c = lax.axis_index("c")
        s = lax.axis_index("s")
        gid = c * NS + s

        cp_idx = pltpu.async_copy(idx_hbm.at[gid], idx_v, sem2)
        cp_msg = pltpu.async_copy(
            msg_hbm.at[pl.ds(gid * per_w, per_w), pl.ds(0, ADIM)], rows_v, sem
        )

        zero16 = jnp.zeros((16,), _f32)

        @pl.loop(0, zrows)
        def _(r):
            zbuf[r, pl.ds(0, 16)] = zero16
            zbuf[r, pl.ds(16, 16)] = zero16

        @pl.loop(0, rows_per_s // zrows)
        def _(i):
            pltpu.sync_copy(zbuf, acc.at[pl.ds(s * rows_per_s + i * zrows, zrows)])

        cp_idx.wait()
        cp_msg.wait()
        plsc.subcore_barrier()

        for j in range(nch):
            pltpu.sync_copy(
                rows_v.at[pl.ds(j * CH, CH)], acc.at[idx_v.at[j]], add=True
            )

        plsc.subcore_barrier()

        pltpu.sync_copy(
            acc.at[pl.ds(s * rows_per_s, rows_per_s)],
            out_hbm.at[pl.ds(c * bn + s * rows_per_s, rows_per_s), pl.ds(0, ADIM)],
        )

    return scatter_kernel


def _msg_body(bond_ref, src_ref, erep_ref, bt2_ref, out_ref):
    bond = bond_ref[0].astype(jnp.bfloat16)
    src = src_ref[:, :ADIM]
    brep = jnp.dot(bond, erep_ref[...], preferred_element_type=_f32).astype(jnp.bfloat16)
    tiled = jnp.concatenate([src] * BDIM, axis=1)
    w = brep * tiled
    out_ref[:, :ADIM] = jnp.dot(w, bt2_ref[...], preferred_element_type=_f32)


def _messages(bond_features, src_atoms, erep, bt2):
    b, e, _ = bond_features.shape
    be = b * e
    rb = 4096
    npb = e // rb
    return pl.pallas_call(
        _msg_body,
        grid=(be // rb,),
        in_specs=[
            pl.BlockSpec((1, rb, BDIM), lambda i: (i // npb, i % npb, 0)),
            pl.BlockSpec((rb, 128), lambda i: (i, 0)),
            pl.BlockSpec((BDIM, BDIM * ADIM), lambda i: (0, 0)),
            pl.BlockSpec((BDIM * ADIM, ADIM), lambda i: (0, 0)),
        ],
        out_specs=pl.BlockSpec((rb, 128), lambda i: (i, 0)),
        out_shape=jax.ShapeDtypeStruct((be, 128), _f32),
    )(bond_features, src_atoms, erep, bt2)


def _gru_body(p0_ref, p1_ref, h_ref, wcat_ref, grkh_ref, bsum_ref, b1h_ref, out_ref):
    x = p0_ref[:, :ADIM] + p1_ref[:, :ADIM]
    h = h_ref[0]
    d = ADIM
    xh = jnp.concatenate([x, h], axis=1)
    m = jnp.dot(xh, wcat_ref[...], preferred_element_type=_f32) + bsum_ref[0:1, :]
    mih = jnp.dot(h, grkh_ref[...], preferred_element_type=_f32) + b1h_ref[0:1, :]
    z = jax.nn.sigmoid(m[:, :d])
    r = jax.nn.sigmoid(m[:, d:2 * d])
    hh = jnp.tanh(m[:, 2 * d:] + (r - 1.0) * mih)
    out_ref[0] = z * h + (1.0 - z) * hh


def _gru(partials, atom_features, gk, grk, bias):
    b, n, _ = atom_features.shape
    bn = b * n
    rb = 4096
    npb = n // rb
    wcat = jnp.concatenate([gk, grk], axis=0)
    grkh = grk[:, 2 * ADIM:]
    bsum = (bias[0] + bias[1]).reshape(1, 3 * ADIM)
    b1h = bias[1, 2 * ADIM:].reshape(1, ADIM)
    return pl.pallas_call(
        _gru_body,
        grid=(bn // rb,),
        in_specs=[
            pl.BlockSpec((rb, 128), lambda i: (i, 0)),
            pl.BlockSpec((rb, 128), lambda i: (bn // rb + i, 0)),
            pl.BlockSpec((1, rb, ADIM), lambda i: (i // npb, i % npb, 0)),
            pl.BlockSpec((2 * ADIM, 3 * ADIM), lambda i: (0, 0)),
            pl.BlockSpec((ADIM, ADIM), lambda i: (0, 0)),
            pl.BlockSpec((1, 3 * ADIM), lambda i: (0, 0)),
            pl.BlockSpec((1, ADIM), lambda i: (0, 0)),
        ],
        out_specs=pl.BlockSpec((1, rb, ADIM), lambda i: (i // npb, i % npb, 0)),
        out_shape=jax.ShapeDtypeStruct((b, n, ADIM), _f32),
    )(partials, partials, atom_features, wcat, grkh, bsum, b1h)


def kernel(atom_features, bond_features, bond_transform, gru_kernel,
           gru_recurrent_kernel, gru_bias, connectivity):
    b, n, d = atom_features.shape
    e = bond_features.shape[1]
    bn, be = b * n, b * e

    atom_bf16 = atom_features.astype(jnp.bfloat16).reshape(bn, d)

    offs = (jnp.arange(b, dtype=jnp.int32) * n)[:, None]
    gsrc = (connectivity[:, :, 0] + offs).reshape(NW, be // NW // CH, CH)
    gtgt = (connectivity[:, :, 1] + offs).reshape(NW, be // NW // CH, CH)

    src_atoms = _make_gather(bn, be)(atom_bf16, gsrc)

    erep = jnp.repeat(jnp.eye(BDIM, dtype=jnp.bfloat16), ADIM, axis=1)
    bt2 = bond_transform.reshape(BDIM * ADIM, ADIM).astype(jnp.bfloat16)
    msgs = _messages(bond_features, src_atoms, erep, bt2)

    partials = _make_scatter(bn, be)(msgs, gtgt)

    return _gru(partials, atom_features,
                gru_kernel, gru_recurrent_kernel, gru_bias)


# speedup vs baseline: 1.3583x; 1.3583x over previous
import functools

import jax
import jax.numpy as jnp
from jax import lax
from jax.experimental import pallas as pl
from jax.experimental.pallas import tpu as pltpu
from jax.experimental.pallas import tpu_sc as plsc

ADIM = 32
BDIM = 16
NC, NS = 2, 16
NW = NC * NS
CH = 128

_f32 = jnp.float32
_SC_PARAMS = pltpu.CompilerParams(use_tc_tiling_on_sc=False)
_SC_PARAMS_TILED = pltpu.CompilerParams(use_tc_tiling_on_sc=True)


def _make_gather(bn: int, be: int):
    per_w = be // NW
    nch = per_w // CH
    mesh = plsc.VectorSubcoreMesh(core_axis_name="c", subcore_axis_name="s")

    @functools.partial(
        pl.kernel,
        out_type=jax.ShapeDtypeStruct((be, 128), _f32),
        mesh=mesh,
        scratch_types=[
            pltpu.VMEM((nch, CH), jnp.int32),
            pltpu.VMEM((per_w, ADIM), _f32),
            pltpu.SemaphoreType.DMA,
        ],
        compiler_params=_SC_PARAMS,
    )
    def gather_kernel(table_hbm, idx_hbm, out_hbm, idx_v, rows_v, sem):
        wid = lax.axis_index("s") * NC + lax.axis_index("c")
        pltpu.sync_copy(idx_hbm.at[wid], idx_v)
        copies = []
        for j in range(nch):
            copies.append(
                pltpu.async_copy(
                    table_hbm.at[idx_v.at[j]],
                    rows_v.at[pl.ds(j * CH, CH)],
                    sem,
                )
            )
        for c in copies:
            c.wait()
        pltpu.sync_copy(
            rows_v, out_hbm.at[pl.ds(wid * per_w, per_w), pl.ds(0, ADIM)]
        )

    return gather_kernel


def _make_scatter(bn: int, be: int):
    per_w = be // NW
    nch = per_w // CH
    rows_per_s = bn // NS
    zrows = 64
    mesh = plsc.VectorSubcoreMesh(core_axis_name="c", subcore_axis_name="s")

    @functools.partial(
        pl.kernel,
        out_type=jax.ShapeDtypeStruct((2 * bn, 128), _f32),
        mesh=mesh,
        scratch_types=[
            pltpu.VMEM((nch, CH), jnp.int32),
            pltpu.VMEM((per_w, ADIM), _f32),
            pltpu.VMEM((zrows, ADIM), _f32),
            pltpu.VMEM_SHARED((bn, ADIM), _f32),
            pltpu.SemaphoreType.DMA,
            pltpu.SemaphoreType.DMA,
        ],
        compiler_params=_SC_PARAMS,
    )
    def scatter_kernel(msg_hbm, idx_hbm, out_hbm, idx_v, rows_v, zbuf, acc, sem, sem2):
        c = lax.axis_index("c")
        s = lax.axis_index("s")
        gid = c * NS + s

        cp_idx = pltpu.async_copy(idx_hbm.at[gid], idx_v, sem2)
        cp_msg = pltpu.async_copy(
            msg_hbm.at[pl.ds(gid * per_w, per_w), pl.ds(0, ADIM)], rows_v, sem
        )

        zero16 = jnp.zeros((16,), _f32)

        @pl.loop(0, zrows)
        def _(r):
            zbuf[r, pl.ds(0, 16)] = zero16
            zbuf[r, pl.ds(16, 16)] = zero16

        @pl.loop(0, rows_per_s // zrows)
        def _(i):
            pltpu.sync_copy(zbuf, acc.at[pl.ds(s * rows_per_s + i * zrows, zrows)])

        cp_idx.wait()
        cp_msg.wait()
        plsc.subcore_barrier()

        for j in range(nch):
            pltpu.sync_copy(
                rows_v.at[pl.ds(j * CH, CH)], acc.at[idx_v.at[j]], add=True
            )

        plsc.subcore_barrier()

        pltpu.sync_copy(
            acc.at[pl.ds(s * rows_per_s, rows_per_s)],
            out_hbm.at[pl.ds(c * bn + s * rows_per_s, rows_per_s), pl.ds(0, ADIM)],
        )

    return scatter_kernel


def _msg_body(bond_ref, src_ref, erep_ref, bt2_ref, out_ref):
    bond = bond_ref[0].astype(jnp.bfloat16)
    src = src_ref[:, :ADIM].astype(jnp.bfloat16)
    brep = jnp.dot(bond, erep_ref[...], preferred_element_type=_f32).astype(jnp.bfloat16)
    tiled = jnp.concatenate([src] * BDIM, axis=1)
    w = brep * tiled
    out_ref[:, :ADIM] = jnp.dot(w, bt2_ref[...], preferred_element_type=_f32)


def _messages(bond_features, src_atoms, erep, bt2):
    b, e, _ = bond_features.shape
    be = b * e
    rb = 4096
    npb = e // rb
    return pl.pallas_call(
        _msg_body,
        grid=(be // rb,),
        in_specs=[
            pl.BlockSpec((1, rb, BDIM), lambda i: (i // npb, i % npb, 0)),
            pl.BlockSpec((rb, 128), lambda i: (i, 0)),
            pl.BlockSpec((BDIM, BDIM * ADIM), lambda i: (0, 0)),
            pl.BlockSpec((BDIM * ADIM, ADIM), lambda i: (0, 0)),
        ],
        out_specs=pl.BlockSpec((rb, 128), lambda i: (i, 0)),
        out_shape=jax.ShapeDtypeStruct((be, 128), _f32),
    )(bond_features, src_atoms, erep, bt2)


def _gru_body(p0_ref, p1_ref, h_ref, wcat_ref, grkh_ref, bsum_ref, b1h_ref, out_ref):
    x = p0_ref[:, :ADIM] + p1_ref[:, :ADIM]
    h = h_ref[0]
    d = ADIM
    xh = jnp.concatenate([x, h], axis=1)
    m = jnp.dot(xh, wcat_ref[...], preferred_element_type=_f32) + bsum_ref[0:1, :]
    mih = jnp.dot(h, grkh_ref[...], preferred_element_type=_f32) + b1h_ref[0:1, :]
    z = jax.nn.sigmoid(m[:, :d])
    r = jax.nn.sigmoid(m[:, d:2 * d])
    hh = jnp.tanh(m[:, 2 * d:] + (r - 1.0) * mih)
    out_ref[0] = z * h + (1.0 - z) * hh


def _gru(partials, atom_features, gk, grk, bias):
    b, n, _ = atom_features.shape
    bn = b * n
    rb = 4096
    npb = n // rb
    wcat = jnp.concatenate([gk, grk], axis=0)
    grkh = grk[:, 2 * ADIM:]
    bsum = (bias[0] + bias[1]).reshape(1, 3 * ADIM)
    b1h = bias[1, 2 * ADIM:].reshape(1, ADIM)
    return pl.pallas_call(
        _gru_body,
        grid=(bn // rb,),
        in_specs=[
            pl.BlockSpec((rb, 128), lambda i: (i, 0)),
            pl.BlockSpec((rb, 128), lambda i: (bn // rb + i, 0)),
            pl.BlockSpec((1, rb, ADIM), lambda i: (i // npb, i % npb, 0)),
            pl.BlockSpec((2 * ADIM, 3 * ADIM), lambda i: (0, 0)),
            pl.BlockSpec((ADIM, ADIM), lambda i: (0, 0)),
            pl.BlockSpec((1, 3 * ADIM), lambda i: (0, 0)),
            pl.BlockSpec((1, ADIM), lambda i: (0, 0)),
        ],
        out_specs=pl.BlockSpec((1, rb, ADIM), lambda i: (i // npb, i % npb, 0)),
        out_shape=jax.ShapeDtypeStruct((b, n, ADIM), _f32),
    )(partials, partials, atom_features, wcat, grkh, bsum, b1h)


def kernel(atom_features, bond_features, bond_transform, gru_kernel,
           gru_recurrent_kernel, gru_bias, connectivity):
    b, n, d = atom_features.shape
    e = bond_features.shape[1]
    bn, be = b * n, b * e

    atom_flat = atom_features.reshape(bn, d)

    offs = (jnp.arange(b, dtype=jnp.int32) * n)[:, None]
    gsrc = (connectivity[:, :, 0] + offs).reshape(NW, be // NW // CH, CH)
    gtgt = (connectivity[:, :, 1] + offs).reshape(NW, be // NW // CH, CH)

    src_atoms = _make_gather(bn, be)(atom_flat, gsrc)

    erep = jnp.repeat(jnp.eye(BDIM, dtype=jnp.bfloat16), ADIM, axis=1)
    bt2 = bond_transform.reshape(BDIM * ADIM, ADIM).astype(jnp.bfloat16)
    msgs = _messages(bond_features, src_atoms, erep, bt2)

    partials = _make_scatter(bn, be)(msgs, gtgt)

    return _gru(partials, atom_features,
                gru_kernel, gru_recurrent_kernel, gru_bias)


# speedup vs baseline: 1.3815x; 1.0171x over previous
import functools

import jax
import jax.numpy as jnp
from jax import lax
from jax.experimental import pallas as pl
from jax.experimental.pallas import tpu as pltpu
from jax.experimental.pallas import tpu_sc as plsc

ADIM = 32
BDIM = 16
NC, NS = 2, 16
NW = NC * NS
CH = 128

_f32 = jnp.float32
_SC_PARAMS = pltpu.CompilerParams(use_tc_tiling_on_sc=False)
_SC_PARAMS_TILED = pltpu.CompilerParams(use_tc_tiling_on_sc=True)


def _make_gather(bn: int, be: int):
    per_w = be // NW
    nch = per_w // CH
    mesh = plsc.VectorSubcoreMesh(core_axis_name="c", subcore_axis_name="s")

    @functools.partial(
        pl.kernel,
        out_type=jax.ShapeDtypeStruct((be, 128), _f32),
        mesh=mesh,
        scratch_types=[
            pltpu.VMEM((nch, CH), jnp.int32),
            pltpu.VMEM((per_w, ADIM), _f32),
            pltpu.SemaphoreType.DMA,
        ],
        compiler_params=_SC_PARAMS,
    )
    def gather_kernel(table_hbm, idx_hbm, out_hbm, idx_v, rows_v, sem):
        wid = lax.axis_index("s") * NC + lax.axis_index("c")
        pltpu.sync_copy(idx_hbm.at[wid], idx_v)
        copies = []
        for j in range(nch):
            copies.append(
                pltpu.async_copy(
                    table_hbm.at[idx_v.at[j]],
                    rows_v.at[pl.ds(j * CH, CH)],
                    sem,
                )
            )
        for c in copies:
            c.wait()
        pltpu.sync_copy(
            rows_v, out_hbm.at[pl.ds(wid * per_w, per_w), pl.ds(0, ADIM)]
        )

    return gather_kernel


def _make_scatter(bn: int, be: int):
    per_w = be // NW
    nch = per_w // CH
    rows_per_s = bn // NS
    zrows = 64
    mesh = plsc.VectorSubcoreMesh(core_axis_name="c", subcore_axis_name="s")

    @functools.partial(
        pl.kernel,
        out_type=jax.ShapeDtypeStruct((bn, 128), _f32),
        mesh=mesh,
        scratch_types=[
            pltpu.VMEM((nch, CH), jnp.int32),
            pltpu.VMEM((per_w, ADIM), _f32),
            pltpu.VMEM((zrows, ADIM), _f32),
            pltpu.VMEM_SHARED((bn, ADIM), _f32),
            pltpu.SemaphoreType.DMA,
            pltpu.SemaphoreType.DMA,
        ],
        compiler_params=_SC_PARAMS,
    )
    def scatter_kernel(msg_hbm, idx_hbm, out_hbm, idx_v, rows_v, zbuf, acc, sem, sem2):
        c = lax.axis_index("c")
        s = lax.axis_index("s")
        gid = c * NS + s

        cp_idx = pltpu.async_copy(idx_hbm.at[gid], idx_v, sem2)
        cp_msg = pltpu.async_copy(
            msg_hbm.at[pl.ds(gid * per_w, per_w), pl.ds(0, ADIM)], rows_v, sem
        )

        zero16 = jnp.zeros((16,), _f32)

        @pl.loop(0, zrows)
        def _(r):
            zbuf[r, pl.ds(0, 16)] = zero16
            zbuf[r, pl.ds(16, 16)] = zero16

        @pl.loop(0, rows_per_s // zrows)
        def _(i):
            pltpu.sync_copy(zbuf, acc.at[pl.ds(s * rows_per_s + i * zrows, zrows)])

        cp_idx.wait()
        cp_msg.wait()
        plsc.subcore_barrier()

        for j in range(nch):
            pltpu.sync_copy(
                rows_v.at[pl.ds(j * CH, CH)], acc.at[idx_v.at[j]], add=True
            )

        plsc.subcore_barrier()

        pltpu.sync_copy(
            acc.at[pl.ds(s * rows_per_s, rows_per_s)],
            out_hbm.at[pl.ds(s * rows_per_s, rows_per_s), pl.ds(c * ADIM, ADIM)],
        )

    return scatter_kernel


def _msg_body(bond_ref, src_ref, erep_ref, bt2_ref, out_ref):
    bond = bond_ref[0].astype(jnp.bfloat16)
    src = src_ref[:, :ADIM].astype(jnp.bfloat16)
    brep = jnp.dot(bond, erep_ref[...], preferred_element_type=_f32).astype(jnp.bfloat16)
    tiled = jnp.concatenate([src] * BDIM, axis=1)
    w = brep * tiled
    out_ref[:, :ADIM] = jnp.dot(w, bt2_ref[...], preferred_element_type=_f32)


def _messages(bond_features, src_atoms, erep, bt2):
    b, e, _ = bond_features.shape
    be = b * e
    rb = 8192
    npb = e // rb
    return pl.pallas_call(
        _msg_body,
        grid=(be // rb,),
        in_specs=[
            pl.BlockSpec((1, rb, BDIM), lambda i: (i // npb, i % npb, 0)),
            pl.BlockSpec((rb, 128), lambda i: (i, 0)),
            pl.BlockSpec((BDIM, BDIM * ADIM), lambda i: (0, 0)),
            pl.BlockSpec((BDIM * ADIM, ADIM), lambda i: (0, 0)),
        ],
        out_specs=pl.BlockSpec((rb, 128), lambda i: (i, 0)),
        out_shape=jax.ShapeDtypeStruct((be, 128), _f32),
    )(bond_features, src_atoms, erep, bt2)


def _gru_body(p_ref, h_ref, wcat_ref, grkh_ref, bsum_ref, b1h_ref, out_ref):
    p = p_ref[...]
    x = p[:, :ADIM] + p[:, ADIM:2 * ADIM]
    h = h_ref[0]
    d = ADIM
    xh = jnp.concatenate([x, h], axis=1)
    m = jnp.dot(xh, wcat_ref[...], preferred_element_type=_f32) + bsum_ref[0:1, :]
    mih = jnp.dot(h, grkh_ref[...], preferred_element_type=_f32) + b1h_ref[0:1, :]
    z = jax.nn.sigmoid(m[:, :d])
    r = jax.nn.sigmoid(m[:, d:2 * d])
    hh = jnp.tanh(m[:, 2 * d:] + (r - 1.0) * mih)
    out_ref[0] = z * h + (1.0 - z) * hh


def _gru(partials, atom_features, gk, grk, bias):
    b, n, _ = atom_features.shape
    bn = b * n
    rb = 4096
    npb = n // rb
    wcat = jnp.concatenate([gk, grk], axis=0)
    grkh = grk[:, 2 * ADIM:]
    bsum = (bias[0] + bias[1]).reshape(1, 3 * ADIM)
    b1h = bias[1, 2 * ADIM:].reshape(1, ADIM)
    return pl.pallas_call(
        _gru_body,
        grid=(bn // rb,),
        in_specs=[
            pl.BlockSpec((rb, 128), lambda i: (i, 0)),
            pl.BlockSpec((1, rb, ADIM), lambda i: (i // npb, i % npb, 0)),
            pl.BlockSpec((2 * ADIM, 3 * ADIM), lambda i: (0, 0)),
            pl.BlockSpec((ADIM, ADIM), lambda i: (0, 0)),
            pl.BlockSpec((1, 3 * ADIM), lambda i: (0, 0)),
            pl.BlockSpec((1, ADIM), lambda i: (0, 0)),
        ],
        out_specs=pl.BlockSpec((1, rb, ADIM), lambda i: (i // npb, i % npb, 0)),
        out_shape=jax.ShapeDtypeStruct((b, n, ADIM), _f32),
    )(partials, atom_features, wcat, grkh, bsum, b1h)


def kernel(atom_features, bond_features, bond_transform, gru_kernel,
           gru_recurrent_kernel, gru_bias, connectivity):
    b, n, d = atom_features.shape
    e = bond_features.shape[1]
    bn, be = b * n, b * e

    atom_flat = atom_features.reshape(bn, d)

    offs = (jnp.arange(b, dtype=jnp.int32) * n)[:, None, None]
    gboth = (connectivity + offs).transpose(2, 0, 1).reshape(2, NW, be // NW // CH, CH)
    gsrc = gboth[0]
    gtgt = gboth[1]

    src_atoms = _make_gather(bn, be)(atom_flat, gsrc)

    erep = jnp.repeat(jnp.eye(BDIM, dtype=jnp.bfloat16), ADIM, axis=1)
    bt2 = bond_transform.reshape(BDIM * ADIM, ADIM).astype(jnp.bfloat16)
    msgs = _messages(bond_features, src_atoms, erep, bt2)

    partials = _make_scatter(bn, be)(msgs, gtgt)

    return _gru(partials, atom_features,
                gru_kernel, gru_recurrent_kernel, gru_bias)
